# Initial kernel scaffold; baseline (speedup 1.0000x reference)
import jax
import jax.numpy as jnp
from jax.experimental import pallas as pl


def kernel(x, edge_index, edge_attr, batch, W_in0, b_in0, g_in0, be_in0, W_in1, b_in1, Wl1, bl1, Wr1, br1, We1, att1, bias1, Wl2, bl2, Wr2, br2, We2, att2, bias2, Wa, ba, Wm0, bm0, gm0, bem0, Wm1, bm1, gm1, bem1, Wm2, bm2, gm2, bem2, Wm3, bm3):
    raise NotImplementedError("write your pallas kernel here")


# speedup vs baseline: 1.0036x; 1.0036x over previous
import jax
import jax.numpy as jnp
from jax.experimental import pallas as pl
from jax.experimental.pallas import tpu as pltpu

N_NODES_C = 50000
N_GRAPHS_C = 16


def _bn(x, g, b):
    mu = jnp.mean(x, axis=0)
    var = jnp.mean((x - mu) ** 2, axis=0)
    return (x - mu) * jax.lax.rsqrt(var + 1e-5) * g + b


def _seg_softmax(a, idx, nseg):
    m = jax.ops.segment_max(a, idx, num_segments=nseg)
    m = jnp.where(jnp.isfinite(m), m, 0.0)
    e = jnp.exp(a - m[idx])
    s = jax.ops.segment_sum(e, idx, num_segments=nseg)
    return e / (s[idx] + 1e-16)


def _gatv2(x, src, dst, edge_attr, Wl, bl, Wr, br, We, att, bias, heads, oc, n):
    xl = (x @ Wl + bl).reshape(n, heads, oc)
    xr = (x @ Wr + br).reshape(n, heads, oc)
    xj = xl[src]
    xi = xr[dst]
    e = (edge_attr @ We).reshape(-1, heads, oc)
    m = jax.nn.leaky_relu(xj + xi + e, 0.2)
    alpha = jnp.sum(m * att[None, :, :], axis=-1)
    alpha = _seg_softmax(alpha, dst, n)
    out = jax.ops.segment_sum(xj * alpha[:, :, None], dst, num_segments=n)
    return out.reshape(n, heads * oc) + bias


def _mlp_kernel(pooled_ref, Wm0_ref, bm0_ref, gm0_ref, bem0_ref,
                Wm1_ref, bm1_ref, gm1_ref, bem1_ref,
                Wm2_ref, bm2_ref, gm2_ref, bem2_ref,
                Wm3_ref, bm3_ref, out_ref):
    def bn_relu(z, g, b):
        mu = jnp.mean(z, axis=0, keepdims=True)
        var = jnp.mean((z - mu) ** 2, axis=0, keepdims=True)
        return jax.nn.relu((z - mu) * jax.lax.rsqrt(var + 1e-5) * g + b)

    z = pooled_ref[...] @ Wm0_ref[...] + bm0_ref[...][None, :]
    z = bn_relu(z, gm0_ref[...][None, :], bem0_ref[...][None, :])
    z = z @ Wm1_ref[...] + bm1_ref[...][None, :]
    z = bn_relu(z, gm1_ref[...][None, :], bem1_ref[...][None, :])
    z = z @ Wm2_ref[...] + bm2_ref[...][None, :]
    z = bn_relu(z, gm2_ref[...][None, :], bem2_ref[...][None, :])
    z = z @ Wm3_ref[...] + bm3_ref[...][None, :]
    out_ref[...] = jax.nn.leaky_relu(z, 0.01)


def kernel(x, edge_index, edge_attr, batch, W_in0, b_in0, g_in0, be_in0, W_in1, b_in1, Wl1, bl1, Wr1, br1, We1, att1, bias1, Wl2, bl2, Wr2, br2, We2, att2, bias2, Wa, ba, Wm0, bm0, gm0, bem0, Wm1, bm1, gm1, bem1, Wm2, bm2, gm2, bem2, Wm3, bm3):
    n = x.shape[0]
    src = edge_index[0]
    dst = edge_index[1]
    h = jax.nn.relu(_bn(x @ W_in0 + b_in0, g_in0, be_in0))
    h = h @ W_in1 + b_in1
    h = _gatv2(h, src, dst, edge_attr, Wl1, bl1, Wr1, br1, We1, att1, bias1, 4, 48, n)
    h = _gatv2(h, src, dst, edge_attr, Wl2, bl2, Wr2, br2, We2, att2, bias2, 4, 96, n)
    att = h @ Wa + ba
    att = _seg_softmax(att, batch, N_GRAPHS_C)
    pooled = jax.ops.segment_sum(h * att, batch, num_segments=N_GRAPHS_C)

    out = pl.pallas_call(
        _mlp_kernel,
        out_shape=jax.ShapeDtypeStruct((N_GRAPHS_C, 2001), jnp.float32),
    )(pooled, Wm0, bm0, gm0, bem0, Wm1, bm1, gm1, bem1,
      Wm2, bm2, gm2, bem2, Wm3, bm3)
    return out


# speedup vs baseline: 3.4500x; 3.4374x over previous
import functools

import jax
import jax.numpy as jnp
from jax import lax
from jax.experimental import pallas as pl
from jax.experimental.pallas import tpu as pltpu
from jax.experimental.pallas import tpu_sc as plsc

N_NODES_C = 50000
N_GRAPHS_C = 16
N_EDGES_C = 800000

_LANES = 16
_IDXW = 128
_BLK = 4
_NTILES = 16
_EPT = 50176
_E_PAD = _EPT * _NTILES
_CHUNKS = _EPT // _IDXW
_NBLK = _CHUNKS // _BLK
_HALF = 25088
_SINK = _HALF
_ACC_ROWS = 25216
_ZROWS = _ACC_ROWS // _NTILES
_HALVES = ((0, _HALF, _HALF // _NTILES),
           (_HALF, N_NODES_C - _HALF, (N_NODES_C - _HALF) // _NTILES))


def _seg_sum_body(ngroups, w_hbm, ia_hbm, ib_hbm, out_hbm,
                  idx_v, buf, zbuf, acc):
    cid = lax.axis_index("c")
    sid = lax.axis_index("s")

    def zz(i, _):
        zbuf[i, :] = jnp.zeros((_LANES,), jnp.float32)
        return 0
    lax.fori_loop(0, _ZROWS, zz, 0)

    ngrp_this = (ngroups + 1) // 2

    def group_body(i, _):
        g = 2 * i + cid

        @pl.when(g < ngroups)
        def _():
            for (base, _hlen, dr), i_hbm in zip(_HALVES, (ia_hbm, ib_hbm)):
                pltpu.sync_copy(zbuf, acc.at[pl.ds(sid * _ZROWS, _ZROWS), :])
                plsc.subcore_barrier()

                def blk_body(b, _):
                    pltpu.sync_copy(
                        w_hbm.at[pl.ds(sid * _CHUNKS + b * _BLK, _BLK),
                                 :, g, :],
                        buf)
                    pltpu.sync_copy(i_hbm.at[sid, pl.ds(b * _BLK, _BLK), :],
                                    idx_v)
                    for j in range(_BLK):
                        pltpu.sync_copy(buf.at[j],
                                        acc.at[idx_v.at[j]],
                                        add=True)
                    return 0
                lax.fori_loop(0, _NBLK, blk_body, 0)

                plsc.subcore_barrier()
                pltpu.sync_copy(
                    acc.at[pl.ds(sid * dr, dr), :],
                    out_hbm.at[pl.ds(base + sid * dr, dr), g, :])
                plsc.subcore_barrier()
        return 0

    lax.fori_loop(0, ngrp_this, group_body, 0)


def _seg_sum_sc(w_ext, idx_a, idx_b, ngroups):
    mesh = plsc.VectorSubcoreMesh(core_axis_name="c", subcore_axis_name="s")
    kfn = pl.kernel(
        functools.partial(_seg_sum_body, ngroups),
        mesh=mesh,
        compiler_params=pltpu.CompilerParams(use_tc_tiling_on_sc=False),
        out_type=jax.ShapeDtypeStruct((N_NODES_C, ngroups, _LANES),
                                      jnp.float32),
        scratch_types=[
            pltpu.VMEM((_BLK, _IDXW), jnp.int32),
            pltpu.VMEM((_BLK, _IDXW, _LANES), jnp.float32),
            pltpu.VMEM((_ZROWS, _LANES), jnp.float32),
            pltpu.VMEM_SHARED((_ACC_ROWS, _LANES), jnp.float32),
        ],
    )
    return kfn(w_ext, idx_a, idx_b)


def _seg_sum_edges(w_flat, idx_a, idx_b, ncols):
    e = w_flat.shape[0]
    ngroups = -(-ncols // _LANES)
    width = ngroups * _LANES
    w_ext = jnp.zeros((_E_PAD, width), jnp.float32)
    w_ext = w_ext.at[:e, :ncols].set(w_flat)
    w_ext = w_ext.reshape(_E_PAD // _IDXW, _IDXW, ngroups, _LANES)
    out = _seg_sum_sc(w_ext, idx_a, idx_b, ngroups)
    return out.reshape(N_NODES_C, width)[:, :ncols]


def _bn(x, g, b):
    mu = jnp.mean(x, axis=0)
    var = jnp.mean((x - mu) ** 2, axis=0)
    return (x - mu) * jax.lax.rsqrt(var + 1e-5) * g + b


def _seg_softmax(a, idx, nseg):
    m = jax.ops.segment_max(a, idx, num_segments=nseg)
    m = jnp.where(jnp.isfinite(m), m, 0.0)
    e = jnp.exp(a - m[idx])
    s = jax.ops.segment_sum(e, idx, num_segments=nseg)
    return e / (s[idx] + 1e-16)


def _gatv2(x, src, dst, idx_a, idx_b, edge_attr, Wl, bl, Wr, br, We, att,
           bias, heads, oc, n):
    xl = (x @ Wl + bl).reshape(n, heads, oc)
    xr = (x @ Wr + br).reshape(n, heads, oc)
    xj = xl[src]
    xi = xr[dst]
    e = (edge_attr @ We).reshape(-1, heads, oc)
    m = jax.nn.leaky_relu(xj + xi + e, 0.2)
    alpha = jnp.sum(m * att[None, :, :], axis=-1)
    ehat = jnp.exp(jnp.clip(alpha, -50.0, 50.0))
    w_flat = jnp.concatenate(
        [(xj * ehat[:, :, None]).reshape(-1, heads * oc), ehat], axis=1)
    num = _seg_sum_edges(w_flat, idx_a, idx_b, heads * oc + heads)
    out = (num[:, :heads * oc].reshape(n, heads, oc)
           / (num[:, heads * oc:, None] + 1e-16))
    return out.reshape(n, heads * oc) + bias


def _mlp_kernel(pooled_ref, Wm0_ref, bm0_ref, gm0_ref, bem0_ref,
                Wm1_ref, bm1_ref, gm1_ref, bem1_ref,
                Wm2_ref, bm2_ref, gm2_ref, bem2_ref,
                Wm3_ref, bm3_ref, out_ref):
    def bn_relu(z, g, b):
        mu = jnp.mean(z, axis=0, keepdims=True)
        var = jnp.mean((z - mu) ** 2, axis=0, keepdims=True)
        return jax.nn.relu((z - mu) * jax.lax.rsqrt(var + 1e-5) * g + b)

    z = pooled_ref[...] @ Wm0_ref[...] + bm0_ref[...][None, :]
    z = bn_relu(z, gm0_ref[...][None, :], bem0_ref[...][None, :])
    z = z @ Wm1_ref[...] + bm1_ref[...][None, :]
    z = bn_relu(z, gm1_ref[...][None, :], bem1_ref[...][None, :])
    z = z @ Wm2_ref[...] + bm2_ref[...][None, :]
    z = bn_relu(z, gm2_ref[...][None, :], bem2_ref[...][None, :])
    z = z @ Wm3_ref[...] + bm3_ref[...][None, :]
    out_ref[...] = jax.nn.leaky_relu(z, 0.01)


def kernel(x, edge_index, edge_attr, batch, W_in0, b_in0, g_in0, be_in0, W_in1, b_in1, Wl1, bl1, Wr1, br1, We1, att1, bias1, Wl2, bl2, Wr2, br2, We2, att2, bias2, Wa, ba, Wm0, bm0, gm0, bem0, Wm1, bm1, gm1, bem1, Wm2, bm2, gm2, bem2, Wm3, bm3):
    n = N_NODES_C
    src = edge_index[0]
    dst = edge_index[1]
    dst_pad = jnp.concatenate(
        [dst, jnp.full((_E_PAD - N_EDGES_C,), N_NODES_C, jnp.int32)])
    idx_a = jnp.where(dst_pad < _HALF, dst_pad, _SINK)
    loc_b = dst_pad - _HALF
    idx_b = jnp.where((loc_b >= 0) & (loc_b < N_NODES_C - _HALF),
                      loc_b, _SINK)
    idx_a = idx_a.reshape(_NTILES, _CHUNKS, _IDXW)
    idx_b = idx_b.reshape(_NTILES, _CHUNKS, _IDXW)

    h = jax.nn.relu(_bn(x @ W_in0 + b_in0, g_in0, be_in0))
    h = h @ W_in1 + b_in1
    h = _gatv2(h, src, dst, idx_a, idx_b, edge_attr, Wl1, bl1, Wr1, br1,
               We1, att1, bias1, 4, 48, n)
    h = _gatv2(h, src, dst, idx_a, idx_b, edge_attr, Wl2, bl2, Wr2, br2,
               We2, att2, bias2, 4, 96, n)
    att = h @ Wa + ba
    att = _seg_softmax(att, batch, N_GRAPHS_C)
    pooled = jax.ops.segment_sum(h * att, batch, num_segments=N_GRAPHS_C)

    out = pl.pallas_call(
        _mlp_kernel,
        out_shape=jax.ShapeDtypeStruct((N_GRAPHS_C, 2001), jnp.float32),
    )(pooled, Wm0, bm0, gm0, bem0, Wm1, bm1, gm1, bem1,
      Wm2, bm2, gm2, bem2, Wm3, bm3)
    return out
